# baseline (speedup 1.0000x reference)
import functools

import jax
import jax.numpy as jnp
from jax import lax
from jax.experimental import pallas as pl

_S = 30.0


def _ldam_body(logits_ref, target_ref, m_ref, out_ref, *, rows, n_classes, batch):
    i = pl.program_id(0)

    x = logits_ref[...]
    t = target_ref[0, 0, :]
    m = m_ref[0, :]

    lane = lax.broadcasted_iota(jnp.int32, (rows, n_classes), 1)
    is_t = lane == t[:, None]

    ones_col = jnp.ones((n_classes, 1), jnp.float32)
    m_b = m[None, :]
    w_b = jnp.exp(-_S * m_b)

    row_max = jnp.max(x, axis=1, keepdims=True)
    e = jnp.exp(x - row_max)

    a = jnp.dot(jnp.where(is_t, x - _S * m_b, 0.0), ones_col,
                preferred_element_type=jnp.float32)
    e_adj = e * jnp.where(is_t, w_b, 1.0)
    se_adj = jnp.dot(e_adj, ones_col, preferred_element_type=jnp.float32)

    nll = row_max + jnp.log(se_adj) - a

    partial = jnp.sum(nll, axis=(0, 1), keepdims=True) * (1.0 / batch)

    @pl.when(i == 0)
    def _():
        out_ref[...] = jnp.zeros_like(out_ref)

    out_ref[...] += partial


def kernel(logits, m_list, target):
    batch, n_classes = logits.shape
    rows = 4096
    grid = batch // rows

    target3 = target.reshape(grid, 1, rows)
    m2 = m_list.reshape(1, n_classes)

    body = functools.partial(_ldam_body, rows=rows, n_classes=n_classes, batch=batch)
    out = pl.pallas_call(
        body,
        grid=(grid,),
        in_specs=[
            pl.BlockSpec((rows, n_classes), lambda i: (i, 0)),
            pl.BlockSpec((1, 1, rows), lambda i: (i, 0, 0)),
            pl.BlockSpec((1, n_classes), lambda i: (0, 0)),
        ],
        out_specs=pl.BlockSpec((1, 1), lambda i: (0, 0)),
        out_shape=jax.ShapeDtypeStruct((1, 1), jnp.float32),
    )(logits, target3, m2)
    return out[0, 0]


# speedup vs baseline: 1.2848x; 1.2848x over previous
import functools

import jax
import jax.numpy as jnp
from jax import lax
from jax.experimental import pallas as pl

_S = 30.0


def _ldam_body(logits_ref, target_ref, m_ref, out_ref, *, rows, n_classes, batch):
    i = pl.program_id(0)

    x = logits_ref[...]
    t = target_ref[0, 0, :]
    m = m_ref[0, :]

    lane = lax.broadcasted_iota(jnp.int32, (rows, n_classes), 1)
    tmask = (lane == t[:, None]).astype(jnp.float32)

    ones_col = jnp.ones((n_classes, 1), jnp.float32)
    m_col = m.reshape(n_classes, 1)

    row_max = jnp.max(x, axis=1, keepdims=True)
    e = jnp.exp(x - row_max)

    l_t = jnp.dot(x * tmask, ones_col, preferred_element_type=jnp.float32)
    m_row = jnp.dot(tmask, m_col, preferred_element_type=jnp.float32)
    se_all = jnp.dot(e, ones_col, preferred_element_type=jnp.float32)

    a = l_t - _S * m_row
    e_t = jnp.exp(l_t - row_max)
    se_adj = se_all - e_t + jnp.exp(a - row_max)
    nll = row_max + jnp.log(se_adj) - a

    partial = jnp.sum(nll, axis=(0, 1), keepdims=True) * (1.0 / batch)

    @pl.when(i == 0)
    def _():
        out_ref[...] = jnp.zeros_like(out_ref)

    out_ref[...] += partial


def kernel(logits, m_list, target):
    batch, n_classes = logits.shape
    rows = 8192
    grid = batch // rows

    target3 = target.reshape(grid, 1, rows)
    m2 = m_list.reshape(1, n_classes)

    body = functools.partial(_ldam_body, rows=rows, n_classes=n_classes, batch=batch)
    out = pl.pallas_call(
        body,
        grid=(grid,),
        in_specs=[
            pl.BlockSpec((rows, n_classes), lambda i: (i, 0)),
            pl.BlockSpec((1, 1, rows), lambda i: (i, 0, 0)),
            pl.BlockSpec((1, n_classes), lambda i: (0, 0)),
        ],
        out_specs=pl.BlockSpec((1, 1), lambda i: (0, 0)),
        out_shape=jax.ShapeDtypeStruct((1, 1), jnp.float32),
    )(logits, target3, m2)
    return out[0, 0]
